# baseline (speedup 1.0000x reference)
import functools

import jax
import jax.numpy as jnp
from jax import lax
from jax.experimental import pallas as pl
from jax.experimental.pallas import tpu as pltpu
from jax.experimental.pallas import tpu_sc as plsc

VOCAB = 1000000
EMBED_DIM = 64
BATCH = 16384
HIST = 50

_NC = 2
_NS = 16
_NW = _NC * _NS
_BPW = BATCH // _NW
_CHUNK = 128
_NCB = _BPW // _CHUNK
_PADD = 128
_TB = 2048


def _make_gather():
    mesh = plsc.VectorSubcoreMesh(core_axis_name="c", subcore_axis_name="s")

    @functools.partial(
        pl.kernel,
        mesh=mesh,
        out_type=jax.ShapeDtypeStruct((HIST, BATCH, _PADD), jnp.float32),
        scratch_types=(
            [pltpu.VMEM((HIST, _NCB, _CHUNK), jnp.int32)]
            + [pltpu.VMEM((_CHUNK, _PADD), jnp.float32) for _ in range(_NCB)]
            + [pltpu.SemaphoreType.DMA for _ in range(2 * _NCB)]
        ),
    )
    def gather_kernel(idx_hbm, table_hbm, out_hbm, idx_v, *bufs_and_sems):
        rows = bufs_and_sems[:_NCB]
        gsem = bufs_and_sems[_NCB:2 * _NCB]
        osem = bufs_and_sems[2 * _NCB:]
        wid = lax.axis_index("s") * _NC + lax.axis_index("c")
        pltpu.sync_copy(idx_hbm.at[wid], idx_v)
        base = wid * _BPW

        def gather_chunk(h, b):
            pltpu.async_copy(table_hbm.at[idx_v.at[h, b]], rows[b], gsem[b])

        for b in range(_NCB):
            gather_chunk(0, b)

        def per_hist(h, carry):
            for b in range(_NCB):
                pltpu.make_async_copy(
                    table_hbm.at[pl.ds(0, _CHUNK)], rows[b], gsem[b]
                ).wait()
                pltpu.async_copy(
                    rows[b],
                    out_hbm.at[h, pl.ds(base + b * _CHUNK, _CHUNK)],
                    osem[b],
                )

                @pl.when(h + 1 < HIST)
                def _():
                    pltpu.make_async_copy(
                        rows[b], out_hbm.at[0, pl.ds(0, _CHUNK)], osem[b]
                    ).wait()
                    gather_chunk(h + 1, b)

            return carry

        lax.fori_loop(0, HIST, per_hist, 0, unroll=False)

        for b in range(_NCB):
            pltpu.make_async_copy(
                rows[b], out_hbm.at[0, pl.ds(0, _CHUNK)], osem[b]
            ).wait()

    return gather_kernel


_gather = _make_gather()


def _transpose_body(in_ref, out_ref):
    eye = jnp.eye(EMBED_DIM, dtype=jnp.float32)
    x = in_ref[0, :, :EMBED_DIM]
    out_ref[0] = jax.lax.dot_general(
        eye, x, (((0,), (1,)), ((), ())),
        preferred_element_type=jnp.float32,
    )


_transpose = pl.pallas_call(
    _transpose_body,
    out_shape=jax.ShapeDtypeStruct((HIST, EMBED_DIM, BATCH), jnp.float32),
    grid=(HIST, BATCH // _TB),
    in_specs=[pl.BlockSpec((1, _TB, _PADD), lambda h, i: (h, i, 0))],
    out_specs=pl.BlockSpec((1, EMBED_DIM, _TB), lambda h, i: (h, 0, i)),
)


def kernel(nodes_batch, features):
    idx = (
        nodes_batch.astype(jnp.int32)
        .T.reshape(HIST, _NW, _NCB * _CHUNK)
        .transpose(1, 0, 2)
        .reshape(_NW, HIST, _NCB, _CHUNK)
    )
    table = jnp.pad(features, ((0, 0), (0, _PADD - EMBED_DIM)))
    mid = _gather(idx, table)
    out = _transpose(mid)
    return out.transpose(2, 0, 1)


# speedup vs baseline: 1.0248x; 1.0248x over previous
import functools

import jax
import jax.numpy as jnp
from jax import lax
from jax.experimental import pallas as pl
from jax.experimental.pallas import tpu as pltpu
from jax.experimental.pallas import tpu_sc as plsc

VOCAB = 1000000
EMBED_DIM = 64
BATCH = 16384
HIST = 50

_NC = 2
_NS = 16
_NW = _NC * _NS
_BPW = BATCH // _NW
_CHUNK = 128
_NCB = _BPW // _CHUNK
_PADD = 128
_L = 16


def _make_gather():
    mesh = plsc.VectorSubcoreMesh(core_axis_name="c", subcore_axis_name="s")

    @functools.partial(
        pl.kernel,
        mesh=mesh,
        compiler_params=pltpu.CompilerParams(needs_layout_passes=False),
        out_type=jax.ShapeDtypeStruct((HIST, EMBED_DIM, BATCH), jnp.float32),
        scratch_types=(
            [pltpu.VMEM((HIST, _NCB, _CHUNK), jnp.int32)]
            + [pltpu.VMEM((_CHUNK, _PADD), jnp.float32) for _ in range(_NCB)]
            + [pltpu.VMEM((EMBED_DIM, _CHUNK), jnp.float32) for _ in range(_NCB)]
            + [pltpu.SemaphoreType.DMA for _ in range(2 * _NCB)]
        ),
    )
    def gather_kernel(idx_hbm, table_hbm, out_hbm, idx_v, *bufs_and_sems):
        rows = bufs_and_sems[:_NCB]
        tbuf = bufs_and_sems[_NCB:2 * _NCB]
        gsem = bufs_and_sems[2 * _NCB:3 * _NCB]
        osem = bufs_and_sems[3 * _NCB:]
        wid = lax.axis_index("s") * _NC + lax.axis_index("c")
        pltpu.sync_copy(idx_hbm.at[wid], idx_v)
        base = wid * _BPW
        lanes = lax.iota(jnp.int32, _L)
        rvecs = [lanes + r0 for r0 in range(0, _CHUNK, _L)]

        def gather_chunk(h, b):
            pltpu.async_copy(table_hbm.at[idx_v.at[h, b]], rows[b], gsem[b])

        for b in range(_NCB):
            gather_chunk(0, b)

        def per_hist(h, carry):
            for b in range(_NCB):
                pltpu.make_async_copy(
                    table_hbm.at[pl.ds(0, _CHUNK)], rows[b], gsem[b]
                ).wait()

                @pl.when(h > 0)
                def _():
                    pltpu.make_async_copy(
                        tbuf[b],
                        out_hbm.at[0, pl.ds(0, EMBED_DIM), pl.ds(0, _CHUNK)],
                        osem[b],
                    ).wait()

                @plsc.parallel_loop(0, EMBED_DIM, unroll=8)
                def transpose_col(c):
                    cvec = jnp.full((_L,), 0, jnp.int32) + c
                    for r0i in range(_CHUNK // _L):
                        vec = plsc.load_gather(rows[b], [rvecs[r0i], cvec])
                        tbuf[b][c, pl.ds(r0i * _L, _L)] = vec

                @pl.when(h + 1 < HIST)
                def _():
                    gather_chunk(h + 1, b)

                pltpu.async_copy(
                    tbuf[b],
                    out_hbm.at[h, pl.ds(0, EMBED_DIM), pl.ds(base + b * _CHUNK, _CHUNK)],
                    osem[b],
                )

            return carry

        lax.fori_loop(0, HIST, per_hist, 0, unroll=False)

        for b in range(_NCB):
            pltpu.make_async_copy(
                tbuf[b], out_hbm.at[0, pl.ds(0, EMBED_DIM), pl.ds(0, _CHUNK)], osem[b]
            ).wait()

    return gather_kernel


_gather = _make_gather()


def kernel(nodes_batch, features):
    idx = (
        nodes_batch.astype(jnp.int32)
        .T.reshape(HIST, _NW, _NCB * _CHUNK)
        .transpose(1, 0, 2)
        .reshape(_NW, HIST, _NCB, _CHUNK)
    )
    table = jnp.pad(features, ((0, 0), (0, _PADD - EMBED_DIM)))
    out = _gather(idx, table)
    return out.transpose(2, 0, 1)
